# baseline (speedup 1.0000x reference)
import functools

import jax
import jax.numpy as jnp
from jax import lax
from jax.experimental import pallas as pl
from jax.experimental.pallas import tpu as pltpu
from jax.experimental.pallas import tpu_sc as plsc

EMB_DIM = 128
NUM_ROWS = 64
NUM_WORKERS = 8
ROWS_PER_WORKER = NUM_ROWS // NUM_WORKERS

_mesh = plsc.VectorSubcoreMesh(
    core_axis_name="c", subcore_axis_name="s", num_cores=1, num_subcores=NUM_WORKERS
)


@functools.partial(
    pl.kernel,
    mesh=_mesh,
    out_type=jax.ShapeDtypeStruct((NUM_ROWS, EMB_DIM), jnp.float32),
    scratch_types=[
        pltpu.VMEM((ROWS_PER_WORKER,), jnp.int32),
        pltpu.VMEM((ROWS_PER_WORKER, EMB_DIM), jnp.float32),
        pltpu.SemaphoreType.DMA,
    ],
)
def _gather_kernel(idx_hbm, table_hbm, out_hbm, idx_v, rows_v, sem):
    wid = lax.axis_index("s")

    @pl.when(wid < NUM_WORKERS)
    def _():
        base = wid * ROWS_PER_WORKER
        pltpu.sync_copy(idx_hbm.at[pl.ds(base, ROWS_PER_WORKER)], idx_v)
        pltpu.async_copy(table_hbm.at[idx_v], rows_v, sem).wait()
        pltpu.sync_copy(rows_v, out_hbm.at[pl.ds(base, ROWS_PER_WORKER)])


def kernel(board_idx, piece_embedding):
    idx = board_idx.reshape(NUM_ROWS).astype(jnp.int32)
    return _gather_kernel(idx, piece_embedding)


# speedup vs baseline: 1.0373x; 1.0373x over previous
import functools

import jax
import jax.numpy as jnp
from jax import lax
from jax.experimental import pallas as pl
from jax.experimental.pallas import tpu as pltpu
from jax.experimental.pallas import tpu_sc as plsc

EMB_DIM = 128
NUM_ROWS = 64

_mesh = plsc.ScalarSubcoreMesh(axis_name="c", num_cores=1)


@functools.partial(
    pl.kernel,
    mesh=_mesh,
    out_type=jax.ShapeDtypeStruct((NUM_ROWS, EMB_DIM), jnp.float32),
    scratch_types=[
        pltpu.SMEM((NUM_ROWS,), jnp.int32),
        pltpu.SemaphoreType.DMA,
    ],
)
def _gather_kernel(idx_hbm, table_hbm, out_hbm, idx_s, sem):
    pltpu.sync_copy(idx_hbm, idx_s)

    def issue(i, carry):
        pltpu.async_copy(table_hbm.at[idx_s[i]], out_hbm.at[i], sem)
        return carry

    lax.fori_loop(0, NUM_ROWS, issue, 0)

    def drain(i, carry):
        pltpu.make_async_copy(table_hbm.at[0], out_hbm.at[i], sem).wait()
        return carry

    lax.fori_loop(0, NUM_ROWS, drain, 0)


def kernel(board_idx, piece_embedding):
    idx = board_idx.reshape(NUM_ROWS).astype(jnp.int32)
    return _gather_kernel(idx, piece_embedding)
